# baseline (speedup 1.0000x reference)
import functools

import jax
import jax.numpy as jnp
from jax import lax
from jax.experimental import pallas as pl
from jax.experimental.pallas import tpu as pltpu
from jax.experimental.pallas import tpu_sc as plsc

_K = 20
_N = 1024
_D = 64
_H = 128
_BIG = 1e10
_EDIM = 2 * _D + 1


def _topk_kernel(xb_ref, idx_ref, d_ref):
    xb = xb_ref[0]
    g = jax.lax.dot_general(xb, xb, (((1,), (1,)), ((), ())),
                            preferred_element_type=jnp.float32,
                            precision=jax.lax.Precision.HIGHEST)
    colid = jax.lax.broadcasted_iota(jnp.int32, (_N, _N), 1)
    rowid = jax.lax.broadcasted_iota(jnp.int32, (_N, _N), 0)
    eye = colid == rowid
    sqj = jnp.sum(jnp.where(eye, g, 0.0), axis=0, keepdims=True)
    d_ref[...] = sqj - 2.0 * g + jnp.where(eye, _BIG, 0.0)

    boff = pl.program_id(0) * _N
    laneid = jax.lax.broadcasted_iota(jnp.int32, (_N, _K), 1)
    idx_ref[0] = jnp.zeros((_N, _K), jnp.int32)

    def body(t, carry):
        d = d_ref[...]
        mn = jnp.min(d, axis=1, keepdims=True)
        idxsel = jnp.min(jnp.where(d <= mn, colid, jnp.int32(2 ** 30)),
                         axis=1, keepdims=True)
        d_ref[...] = jnp.where(colid == idxsel, jnp.float32(3e38), d)
        idx_ref[0] = jnp.where(laneid == t, idxsel + boff, idx_ref[0])
        return carry

    jax.lax.fori_loop(0, _K, body, 0)


def _topk_indices(x):
    B = x.shape[0]
    return pl.pallas_call(
        _topk_kernel,
        grid=(B,),
        in_specs=[pl.BlockSpec((1, _N, _D), lambda b: (b, 0, 0))],
        out_specs=pl.BlockSpec((1, _N, _K), lambda b: (b, 0, 0)),
        out_shape=jax.ShapeDtypeStruct((B, _N, _K), jnp.int32),
        scratch_shapes=[pltpu.VMEM((_N, _N), jnp.float32)],
    )(x)


def _sc_gather(table, idx_flat):
    E = idx_flat.shape[0]
    W = table.shape[1]
    info = plsc.get_sparse_core_info()
    nc, ns = info.num_cores, info.num_subcores
    nw = nc * ns
    ch = 128
    e_per_w = E // nw
    n_chunks = e_per_w // ch
    assert e_per_w * nw == E and n_chunks * ch == e_per_w

    mesh = plsc.VectorSubcoreMesh(core_axis_name="c", subcore_axis_name="s")

    @functools.partial(
        pl.kernel, mesh=mesh,
        out_type=jax.ShapeDtypeStruct((E, W), jnp.float32),
        scratch_types=[
            pltpu.VMEM((ch,), jnp.int32),
            pltpu.VMEM((ch,), jnp.int32),
            pltpu.VMEM((ch, W), jnp.float32),
            pltpu.VMEM((ch, W), jnp.float32),
            pltpu.SemaphoreType.DMA,
            pltpu.SemaphoreType.DMA,
        ],
    )
    def gather_k(table_hbm, idx_hbm, out_hbm, idx0, idx1, rows0, rows1,
                 sem0, sem1):
        wid = lax.axis_index("s") * nc + lax.axis_index("c")
        base = wid * e_per_w
        idxv = (idx0, idx1)
        rowsv = (rows0, rows1)
        sems = (sem0, sem1)
        pltpu.sync_copy(idx_hbm.at[pl.ds(base, ch)], idx0)
        cp0 = pltpu.async_copy(table_hbm.at[idx0], rows0, sem0)
        copies = [cp0]
        for c in range(n_chunks):
            s = c % 2
            if c + 1 < n_chunks:
                sn = (c + 1) % 2
                pltpu.sync_copy(idx_hbm.at[pl.ds(base + (c + 1) * ch, ch)],
                                idxv[sn])
                copies.append(
                    pltpu.async_copy(table_hbm.at[idxv[sn]], rowsv[sn],
                                     sems[sn]))
            copies[c].wait()
            pltpu.sync_copy(rowsv[s], out_hbm.at[pl.ds(base + c * ch, ch)])

    return gather_k(table, idx_flat)


def _mlp_kernel(xb_ref, nbr_ref, w1x_ref, w1d_ref, w1e_ref, csum_ref, b1_ref,
                w2_ref, b2_ref, out_ref, ax_ref):
    t = pl.program_id(1)
    xb = xb_ref[0]

    @pl.when(t == 0)
    def _init():
        ax_ref[...] = jax.lax.dot_general(
            xb, w1x_ref[...], (((1,), (0,)), ((), ())),
            preferred_element_type=jnp.float32)
        out_ref[0] = jnp.full((_N, _D), -jnp.inf, dtype=jnp.float32)

    nbr = nbr_ref[0, 0][:, :_D]
    diff = nbr - xb
    e = jnp.sum(diff * diff, axis=1, keepdims=True)
    sx = jnp.sum(xb, axis=1, keepdims=True)
    ssx = jnp.sum(xb * xb, axis=1, keepdims=True)
    mean = (sx + jnp.sum(diff, axis=1, keepdims=True) + e) * (1.0 / _EDIM)
    msq = (ssx + e + e * e) * (1.0 / _EDIM)
    r = jax.lax.rsqrt(msq - mean * mean + 1e-6)
    ad = jax.lax.dot_general(diff, w1d_ref[...], (((1,), (0,)), ((), ())),
                             preferred_element_type=jnp.float32)
    h1 = (r * (ax_ref[...] + ad + e * w1e_ref[...])
          - (mean * r) * csum_ref[...] + b1_ref[...])
    h = h1 * (1.0 / (1.0 + jnp.exp(-h1)))
    h2 = jax.lax.dot_general(h, w2_ref[...], (((1,), (0,)), ((), ())),
                             preferred_element_type=jnp.float32) + b2_ref[...]
    out_ref[0] = jnp.maximum(out_ref[0], h2)


def _edge_mlp(x, nbr, w1x, w1d, w1e, csum, b1f, W2, b2r):
    B = x.shape[0]
    return pl.pallas_call(
        _mlp_kernel,
        grid=(B, _K),
        in_specs=[
            pl.BlockSpec((1, _N, _D), lambda b, t: (b, 0, 0)),
            pl.BlockSpec((1, 1, _N, _H), lambda b, t: (b, t, 0, 0)),
            pl.BlockSpec((_D, _H), lambda b, t: (0, 0)),
            pl.BlockSpec((_D, _H), lambda b, t: (0, 0)),
            pl.BlockSpec((1, _H), lambda b, t: (0, 0)),
            pl.BlockSpec((1, _H), lambda b, t: (0, 0)),
            pl.BlockSpec((1, _H), lambda b, t: (0, 0)),
            pl.BlockSpec((_H, _D), lambda b, t: (0, 0)),
            pl.BlockSpec((1, _D), lambda b, t: (0, 0)),
        ],
        out_specs=pl.BlockSpec((1, _N, _D), lambda b, t: (b, 0, 0)),
        out_shape=jax.ShapeDtypeStruct((B, _N, _D), jnp.float32),
        scratch_shapes=[pltpu.VMEM((_N, _H), jnp.float32)],
    )(x, nbr, w1x, w1d, w1e, csum, b1f, W2, b2r)


@jax.jit
def kernel(x, ln_scale, ln_bias, W1, b1, W2, b2):
    B, N, D = x.shape
    w1s = ln_scale[:, None] * W1
    b1f = (b1 + ln_bias @ W1)[None, :]
    csum = jnp.sum(w1s, axis=0)[None, :]
    w1x = w1s[:D]
    w1d = w1s[D:2 * D]
    w1e = w1s[2 * D:2 * D + 1]

    idx = _topk_indices(x)
    idx_btn = jnp.swapaxes(idx, 1, 2).reshape(-1)
    table = jnp.concatenate(
        [x.reshape(B * N, D), jnp.zeros((B * N, _H - D), jnp.float32)], axis=1)
    nbr_flat = _sc_gather(table, idx_btn)
    nbr = nbr_flat.reshape(B, _K, N, _H)
    return _edge_mlp(x, nbr, w1x, w1d, w1e, csum, b1f, W2, b2[None, :])
